# baseline (speedup 1.0000x reference)
import functools

import jax
import jax.numpy as jnp
from jax import lax
from jax.experimental import pallas as pl
from jax.experimental.pallas import tpu as pltpu
from jax.experimental.pallas import tpu_sc as plsc

VOCAB = 30000
EMB = 128
B = 4
S = 2048
EPS = 1e-12

NC = 2
NS = 16
NW = NC * NS
TOK = B * S
TPW = TOK // NW
IDXW = 128
NIDX = TPW // IDXW


@functools.partial(
    pl.kernel,
    out_type=jax.ShapeDtypeStruct((TOK, EMB), jnp.float32),
    mesh=plsc.VectorSubcoreMesh(core_axis_name="c", subcore_axis_name="s"),
    scratch_types=[
        pltpu.VMEM((NIDX, IDXW), jnp.int32),
        pltpu.VMEM((TPW, EMB), jnp.float32),
        pltpu.SemaphoreType.DMA,
        pltpu.SemaphoreType.DMA,
        pltpu.SemaphoreType.DMA,
    ],
)
def _gather(ids_hbm, w_hbm, out_hbm, idx_v, rows_v, gsem0, gsem1, wsem):
    cid = lax.axis_index("c")
    sid = lax.axis_index("s")
    wid = sid * NC + cid
    base = wid * TPW

    pltpu.sync_copy(ids_hbm.at[pl.ds(wid * NIDX, NIDX)], idx_v)

    gsems = [gsem0, gsem1]
    gcps = [
        pltpu.async_copy(w_hbm.at[idx_v.at[j]],
                         rows_v.at[pl.ds(j * IDXW, IDXW)], gsems[j])
        for j in range(NIDX)
    ]
    wcps = []
    for j in range(NIDX):
        gcps[j].wait()
        wcps.append(pltpu.async_copy(
            rows_v.at[pl.ds(j * IDXW, IDXW)],
            out_hbm.at[pl.ds(base + j * IDXW, IDXW)], wsem))
    for cp in wcps:
        cp.wait()


def _ln_body(mid_ref, pos_ref, tte_ref, g_ref, b_ref, o_ref):
    x = mid_ref[...] + pos_ref[...] + tte_ref[0:1, :]
    avg = jnp.full((EMB, EMB), 1.0 / EMB, dtype=jnp.float32)
    m = jax.lax.dot(x, avg, precision=jax.lax.Precision.HIGHEST)
    d = x - m
    var = jax.lax.dot(d * d, avg, precision=jax.lax.Precision.HIGHEST)
    o_ref[...] = (d * lax.rsqrt(var + EPS) * g_ref[...] + b_ref[...])[None]


_ln_call = pl.pallas_call(
    _ln_body,
    out_shape=jax.ShapeDtypeStruct((B, S, EMB), jnp.float32),
    grid=(B,),
    in_specs=[
        pl.BlockSpec((S, EMB), lambda i: (i, 0)),
        pl.BlockSpec((S, EMB), lambda i: (0, 0)),
        pl.BlockSpec((2, EMB), lambda i: (0, 0)),
        pl.BlockSpec((1, EMB), lambda i: (0, 0)),
        pl.BlockSpec((1, EMB), lambda i: (0, 0)),
    ],
    out_specs=pl.BlockSpec((1, S, EMB), lambda i: (i, 0, 0)),
)


def kernel(input_ids, weight, token_type_embeddings, position_embeddings,
           ln_gamma, ln_beta):
    mid = _gather(input_ids.astype(jnp.int32).reshape(TOK // IDXW, IDXW),
                  weight)
    return _ln_call(mid,
                    position_embeddings,
                    token_type_embeddings,
                    ln_gamma.reshape(1, EMB),
                    ln_beta.reshape(1, EMB))


# speedup vs baseline: 1.3160x; 1.3160x over previous
import functools

import jax
import jax.numpy as jnp
from jax import lax
from jax.experimental import pallas as pl
from jax.experimental.pallas import tpu as pltpu
from jax.experimental.pallas import tpu_sc as plsc

VOCAB = 30000
EMB = 128
B = 4
S = 2048
EPS = 1e-12

NC = 2
NS = 16
NW = NC * NS
TOK = B * S
TPW = TOK // NW
IDXW = 128
NIDX = TPW // IDXW


@functools.partial(
    pl.kernel,
    out_type=jax.ShapeDtypeStruct((TOK, EMB), jnp.float32),
    mesh=plsc.VectorSubcoreMesh(core_axis_name="c", subcore_axis_name="s"),
    scratch_types=[
        pltpu.VMEM((NIDX, IDXW), jnp.int32),
        pltpu.VMEM((TPW, EMB), jnp.float32),
        pltpu.SemaphoreType.DMA,
        pltpu.SemaphoreType.DMA,
        pltpu.SemaphoreType.DMA,
    ],
)
def _gather(ids_hbm, w_hbm, out_hbm, idx_v, rows_v, gsem0, gsem1, wsem):
    cid = lax.axis_index("c")
    sid = lax.axis_index("s")
    wid = sid * NC + cid
    base = wid * TPW

    pltpu.sync_copy(ids_hbm.at[pl.ds(wid * NIDX, NIDX)], idx_v)

    gsems = [gsem0, gsem1]
    gcps = [
        pltpu.async_copy(w_hbm.at[idx_v.at[j]],
                         rows_v.at[pl.ds(j * IDXW, IDXW)], gsems[j])
        for j in range(NIDX)
    ]
    wcps = []
    for j in range(NIDX):
        gcps[j].wait()
        wcps.append(pltpu.async_copy(
            rows_v.at[pl.ds(j * IDXW, IDXW)],
            out_hbm.at[pl.ds(base + j * IDXW, IDXW)], wsem))
    for cp in wcps:
        cp.wait()


def _ln_body(mid_ref, pos_ref, tte_ref, g_ref, b_ref, o_ref):
    x = mid_ref[...] + pos_ref[...] + tte_ref[0:1, :]
    m = jnp.mean(x, axis=-1, keepdims=True)
    d = x - m
    var = jnp.mean(d * d, axis=-1, keepdims=True)
    o_ref[...] = (d * lax.rsqrt(var + EPS) * g_ref[...] + b_ref[...])[None]


_ln_call = pl.pallas_call(
    _ln_body,
    out_shape=jax.ShapeDtypeStruct((B, S, EMB), jnp.float32),
    grid=(B,),
    in_specs=[
        pl.BlockSpec((S, EMB), lambda i: (i, 0)),
        pl.BlockSpec((S, EMB), lambda i: (0, 0)),
        pl.BlockSpec((2, EMB), lambda i: (0, 0)),
        pl.BlockSpec((1, EMB), lambda i: (0, 0)),
        pl.BlockSpec((1, EMB), lambda i: (0, 0)),
    ],
    out_specs=pl.BlockSpec((1, S, EMB), lambda i: (i, 0, 0)),
)


def kernel(input_ids, weight, token_type_embeddings, position_embeddings,
           ln_gamma, ln_beta):
    mid = _gather(input_ids.astype(jnp.int32).reshape(TOK // IDXW, IDXW),
                  weight)
    return _ln_call(mid,
                    position_embeddings,
                    token_type_embeddings,
                    ln_gamma.reshape(1, EMB),
                    ln_beta.reshape(1, EMB))


# speedup vs baseline: 1.3217x; 1.0043x over previous
import functools

import jax
import jax.numpy as jnp
from jax import lax
from jax.experimental import pallas as pl
from jax.experimental.pallas import tpu as pltpu
from jax.experimental.pallas import tpu_sc as plsc

VOCAB = 30000
EMB = 128
B = 4
S = 2048
EPS = 1e-12

NC = 2
NS = 16
NW = NC * NS
TOK = B * S
TPW = TOK // NW
IDXW = 128
NIDX = TPW // IDXW


@functools.partial(
    pl.kernel,
    out_type=jax.ShapeDtypeStruct((TOK, EMB), jnp.float32),
    mesh=plsc.VectorSubcoreMesh(core_axis_name="c", subcore_axis_name="s"),
    scratch_types=[
        pltpu.VMEM((NIDX, IDXW), jnp.int32),
        pltpu.VMEM((TPW, EMB), jnp.float32),
        pltpu.SemaphoreType.DMA,
        pltpu.SemaphoreType.DMA,
        pltpu.SemaphoreType.DMA,
    ],
)
def _gather(ids_hbm, w_hbm, out_hbm, idx_v, rows_v, gsem0, gsem1, wsem):
    cid = lax.axis_index("c")
    sid = lax.axis_index("s")
    wid = sid * NC + cid
    base = wid * TPW

    pltpu.sync_copy(ids_hbm.at[pl.ds(wid * NIDX, NIDX)], idx_v)

    gsems = [gsem0, gsem1]
    gcps = [
        pltpu.async_copy(w_hbm.at[idx_v.at[j]],
                         rows_v.at[pl.ds(j * IDXW, IDXW)], gsems[j])
        for j in range(NIDX)
    ]
    wcps = []
    for j in range(NIDX):
        gcps[j].wait()
        wcps.append(pltpu.async_copy(
            rows_v.at[pl.ds(j * IDXW, IDXW)],
            out_hbm.at[pl.ds(base + j * IDXW, IDXW)], wsem))
    for cp in wcps:
        cp.wait()


def _ln_body(mid_ref, pos_ref, tte_ref, g_ref, b_ref, o_ref):
    x = mid_ref[...] + pos_ref[...] + tte_ref[0:1, :]
    m = jnp.mean(x, axis=-1, keepdims=True)
    q = jnp.mean(x * x, axis=-1, keepdims=True)
    var = q - m * m
    o_ref[...] = ((x - m) * lax.rsqrt(var + EPS) * g_ref[...] + b_ref[...])[None]


_ln_call = pl.pallas_call(
    _ln_body,
    out_shape=jax.ShapeDtypeStruct((B, S, EMB), jnp.float32),
    grid=(B,),
    in_specs=[
        pl.BlockSpec((S, EMB), lambda i: (i, 0)),
        pl.BlockSpec((S, EMB), lambda i: (0, 0)),
        pl.BlockSpec((2, EMB), lambda i: (0, 0)),
        pl.BlockSpec((1, EMB), lambda i: (0, 0)),
        pl.BlockSpec((1, EMB), lambda i: (0, 0)),
    ],
    out_specs=pl.BlockSpec((1, S, EMB), lambda i: (i, 0, 0)),
)


def kernel(input_ids, weight, token_type_embeddings, position_embeddings,
           ln_gamma, ln_beta):
    mid = _gather(input_ids.astype(jnp.int32).reshape(TOK // IDXW, IDXW),
                  weight)
    return _ln_call(mid,
                    position_embeddings,
                    token_type_embeddings,
                    ln_gamma.reshape(1, EMB),
                    ln_beta.reshape(1, EMB))
